# Initial kernel scaffold; baseline (speedup 1.0000x reference)
import jax
import jax.numpy as jnp
from jax.experimental import pallas as pl


def kernel(x, conv_w, conv_b, bn_gamma, bn_beta, bn_mean, bn_var):
    raise NotImplementedError("write your pallas kernel here")


# speedup vs baseline: 2.5285x; 2.5285x over previous
import functools

import jax
import jax.numpy as jnp
from jax.experimental import pallas as pl
from jax.experimental.pallas import tpu as pltpu

LANES = 128


def _src_coords(n_in, n_out):
    if n_out == 1:
        return jnp.zeros((1,), jnp.float32)
    return jnp.arange(n_out, dtype=jnp.float32) * ((n_in - 1) / (n_out - 1))


def _interp_matrix(n_in, n_out):
    src = _src_coords(n_in, n_out)
    i0 = jnp.clip(jnp.floor(src).astype(jnp.int32), 0, n_in - 1)
    i1 = jnp.minimum(i0 + 1, n_in - 1)
    frac = src - i0.astype(jnp.float32)
    rows = jnp.arange(n_out)
    a = jnp.zeros((n_out, n_in), jnp.float32)
    a = a.at[rows, i0].add(1.0 - frac)
    a = a.at[rows, i1].add(frac)
    return a


def _fused_kernel(x_ref, awt_ref, ahcat_ref, w_ref, shift_ref, o_ref,
                  xw_ref, z_ref, *, cin, cout, hin, hout, wout, kh, kw):

    xw_ref[...] = jnp.dot(x_ref[0], awt_ref[...],
                          preferred_element_type=jnp.float32)

    for co in range(cout):
        for dh in range(kh):
            acc = jnp.zeros((hin, wout), jnp.float32)
            for ci in range(cin):
                for dw in range(kw):
                    widx = ((co * kh + dh) * cin + ci) * kw + dw
                    win = xw_ref[pl.ds(ci * hin, hin), pl.ds(dw, wout)]
                    acc = acc + w_ref[widx] * win
            z_ref[pl.ds((co * kh + dh) * hin, hin), :] = acc

    for co in range(cout):
        zc = z_ref[pl.ds(co * kh * hin, kh * hin), :]
        o_ref[0, co] = (jnp.dot(ahcat_ref[...], zc,
                                preferred_element_type=jnp.float32)
                        + shift_ref[co]).astype(o_ref.dtype)


def _upsample_conv_bn(x, size, conv_w, conv_b, bn_gamma, bn_beta,
                      bn_mean, bn_var, eps=1e-5):
    N, Cin, Hin, Win = x.shape
    Hout, Wout = size
    Cout, Cin_w, KH, KW = conv_w.shape
    assert Cin_w == Cin
    PH, PW = KH // 2, KW // 2
    Wout_p = ((Wout + LANES - 1) // LANES) * LANES
    Wp = Wout_p + 2 * PW

    aw = _interp_matrix(Win, Wout)
    awt = jnp.pad(aw, ((PW, PW + (Wout_p - Wout)), (0, 0))).T

    ah_pad = jnp.pad(_interp_matrix(Hin, Hout), ((PH, PH), (0, 0)))
    ahcat = jnp.concatenate([ah_pad[dh:dh + Hout, :] for dh in range(KH)],
                            axis=1)

    scale = bn_gamma / jnp.sqrt(bn_var + eps)
    w_eff = conv_w.astype(jnp.float32) * scale[:, None, None, None]
    shift = ((conv_b - bn_mean) * scale + bn_beta).astype(jnp.float32)
    w_flat = w_eff.transpose(0, 2, 1, 3).reshape(-1)

    x2d = x.reshape(N, Cin * Hin, Win)

    kernel_fn = functools.partial(
        _fused_kernel, cin=Cin, cout=Cout, hin=Hin, hout=Hout,
        wout=Wout_p, kh=KH, kw=KW)

    flops = N * (2 * Cin * Hin * Win * Wp
                 + 2 * Cout * KH * Cin * KW * Hin * Wout_p
                 + 2 * Cout * Hout * KH * Hin * Wout_p)
    bytes_accessed = 4 * (x2d.size + awt.size + ahcat.size + w_flat.size
                          + shift.size + N * Cout * Hout * Wout_p)

    out = pl.pallas_call(
        kernel_fn,
        out_shape=jax.ShapeDtypeStruct((N, Cout, Hout, Wout_p), x.dtype),
        grid_spec=pltpu.PrefetchScalarGridSpec(
            num_scalar_prefetch=0,
            grid=(N,),
            in_specs=[
                pl.BlockSpec((1, Cin * Hin, Win), lambda n: (n, 0, 0)),
                pl.BlockSpec((Win, Wp), lambda n: (0, 0)),
                pl.BlockSpec((Hout, KH * Hin), lambda n: (0, 0)),
                pl.BlockSpec(memory_space=pltpu.MemorySpace.SMEM),
                pl.BlockSpec(memory_space=pltpu.MemorySpace.SMEM),
            ],
            out_specs=pl.BlockSpec((1, Cout, Hout, Wout_p),
                                   lambda n: (n, 0, 0, 0)),
            scratch_shapes=[
                pltpu.VMEM((Cin * Hin, Wp), jnp.float32),
                pltpu.VMEM((Cout * KH * Hin, Wout_p), jnp.float32),
            ],
        ),
        compiler_params=pltpu.CompilerParams(
            dimension_semantics=("parallel",),
            vmem_limit_bytes=32 * 1024 * 1024),
        cost_estimate=pl.CostEstimate(
            flops=flops, transcendentals=0, bytes_accessed=bytes_accessed),
    )(x2d, awt, ahcat, w_flat, shift)

    return out[..., :Wout]


def kernel(x, conv_w, conv_b, bn_gamma, bn_beta, bn_mean, bn_var):
    return _upsample_conv_bn(x, (128, 128), conv_w, conv_b,
                             bn_gamma, bn_beta, bn_mean, bn_var)


# speedup vs baseline: 12.0820x; 4.7783x over previous
import functools

import jax
import jax.numpy as jnp
from jax.experimental import pallas as pl
from jax.experimental.pallas import tpu as pltpu

LANES = 128


def _src_coords(n_in, n_out):
    if n_out == 1:
        return jnp.zeros((1,), jnp.float32)
    return jnp.arange(n_out, dtype=jnp.float32) * ((n_in - 1) / (n_out - 1))


def _interp_matrix(n_in, n_out):
    src = _src_coords(n_in, n_out)
    i0 = jnp.clip(jnp.floor(src).astype(jnp.int32), 0, n_in - 1)
    i1 = jnp.minimum(i0 + 1, n_in - 1)
    frac = src - i0.astype(jnp.float32)
    rows = jnp.arange(n_out)
    a = jnp.zeros((n_out, n_in), jnp.float32)
    a = a.at[rows, i0].add(1.0 - frac)
    a = a.at[rows, i1].add(frac)
    return a


def _fused_kernel(x_ref, awt5_ref, ahcat_ref, w_ref, shift_ref, o_ref,
                  xw_ref, z_ref, *, cin, cout, hin, hout, wout, kh, kw):

    xw_ref[...] = jnp.dot(x_ref[0], awt5_ref[...],
                          preferred_element_type=jnp.float32)

    for co in range(cout):
        accs = [jnp.zeros((hin, wout), jnp.float32) for _ in range(kh)]
        for ci in range(cin):
            for dw in range(kw):
                win = xw_ref[pl.ds(ci * hin, hin),
                             pl.ds(dw * wout, wout)]
                for dh in range(kh):
                    widx = ((co * kh + dh) * cin + ci) * kw + dw
                    accs[dh] = accs[dh] + w_ref[widx] * win
        for dh in range(kh):
            z_ref[pl.ds(dh * hin, hin), pl.ds(co * wout, wout)] = accs[dh]

    big = jnp.dot(ahcat_ref[...], z_ref[...],
                  preferred_element_type=jnp.float32)
    for co in range(cout):
        o_ref[0, co] = (big[:, co * wout:(co + 1) * wout]
                        + shift_ref[co]).astype(o_ref.dtype)


def _upsample_conv_bn(x, size, conv_w, conv_b, bn_gamma, bn_beta,
                      bn_mean, bn_var, eps=1e-5):
    N, Cin, Hin, Win = x.shape
    Hout, Wout = size
    Cout, Cin_w, KH, KW = conv_w.shape
    assert Cin_w == Cin
    PH, PW = KH // 2, KW // 2
    Wout_p = ((Wout + LANES - 1) // LANES) * LANES
    Wp = Wout_p + 2 * PW

    aw = _interp_matrix(Win, Wout)
    aw_pad = jnp.pad(aw, ((PW, PW + (Wout_p - Wout)), (0, 0)))
    awt5 = jnp.concatenate([aw_pad[dw:dw + Wout_p, :] for dw in range(KW)],
                           axis=0).T

    ah_pad = jnp.pad(_interp_matrix(Hin, Hout), ((PH, PH), (0, 0)))
    ahcat = jnp.concatenate([ah_pad[dh:dh + Hout, :] for dh in range(KH)],
                            axis=1)

    scale = bn_gamma / jnp.sqrt(bn_var + eps)
    w_eff = conv_w.astype(jnp.float32) * scale[:, None, None, None]
    shift = ((conv_b - bn_mean) * scale + bn_beta).astype(jnp.float32)
    w_flat = w_eff.transpose(0, 2, 1, 3).reshape(-1)

    x2d = x.reshape(N, Cin * Hin, Win)

    kernel_fn = functools.partial(
        _fused_kernel, cin=Cin, cout=Cout, hin=Hin, hout=Hout,
        wout=Wout_p, kh=KH, kw=KW)

    flops = N * (2 * Cin * Hin * Win * KW * Wout_p
                 + 2 * Cout * KH * Cin * KW * Hin * Wout_p
                 + 2 * Cout * Hout * KH * Hin * Wout_p)
    bytes_accessed = 4 * (x2d.size + awt5.size + ahcat.size + w_flat.size
                          + shift.size + N * Cout * Hout * Wout_p)

    out = pl.pallas_call(
        kernel_fn,
        out_shape=jax.ShapeDtypeStruct((N, Cout, Hout, Wout_p), x.dtype),
        grid_spec=pltpu.PrefetchScalarGridSpec(
            num_scalar_prefetch=0,
            grid=(N,),
            in_specs=[
                pl.BlockSpec((1, Cin * Hin, Win), lambda n: (n, 0, 0)),
                pl.BlockSpec((Win, KW * Wout_p), lambda n: (0, 0)),
                pl.BlockSpec((Hout, KH * Hin), lambda n: (0, 0)),
                pl.BlockSpec(memory_space=pltpu.MemorySpace.SMEM),
                pl.BlockSpec(memory_space=pltpu.MemorySpace.SMEM),
            ],
            out_specs=pl.BlockSpec((1, Cout, Hout, Wout_p),
                                   lambda n: (n, 0, 0, 0)),
            scratch_shapes=[
                pltpu.VMEM((Cin * Hin, KW * Wout_p), jnp.float32),
                pltpu.VMEM((KH * Hin, Cout * Wout_p), jnp.float32),
            ],
        ),
        compiler_params=pltpu.CompilerParams(
            dimension_semantics=("parallel",),
            vmem_limit_bytes=32 * 1024 * 1024),
        cost_estimate=pl.CostEstimate(
            flops=flops, transcendentals=0, bytes_accessed=bytes_accessed),
    )(x2d, awt5, ahcat, w_flat, shift)

    return out[..., :Wout]


def kernel(x, conv_w, conv_b, bn_gamma, bn_beta, bn_mean, bn_var):
    return _upsample_conv_bn(x, (128, 128), conv_w, conv_b,
                             bn_gamma, bn_beta, bn_mean, bn_var)


# speedup vs baseline: 40.8597x; 3.3819x over previous
import functools

import jax
import jax.numpy as jnp
from jax.experimental import pallas as pl
from jax.experimental.pallas import tpu as pltpu

LANES = 128


def _src_coords(n_in, n_out):
    if n_out == 1:
        return jnp.zeros((1,), jnp.float32)
    return jnp.arange(n_out, dtype=jnp.float32) * ((n_in - 1) / (n_out - 1))


def _interp_matrix(n_in, n_out):
    src = _src_coords(n_in, n_out)
    i0 = jnp.clip(jnp.floor(src).astype(jnp.int32), 0, n_in - 1)
    i1 = jnp.minimum(i0 + 1, n_in - 1)
    frac = src - i0.astype(jnp.float32)
    rows = jnp.arange(n_out)
    a = jnp.zeros((n_out, n_in), jnp.float32)
    a = a.at[rows, i0].add(1.0 - frac)
    a = a.at[rows, i1].add(frac)
    return a


def _fused_kernel(x_ref, rw_ref, ahcat_ref, shift_ref, o_ref,
                  b_ref, z_ref, *, bs, cin, cout, hin, hout, wout, kh, kw):

    nco = cout * wout

    b_ref[...] = jnp.dot(x_ref[...].reshape(bs * hin, -1), rw_ref[...],
                         preferred_element_type=jnp.float32)

    for s in range(bs):
        for dh in range(kh):
            z_ref[pl.ds(dh * hin, hin), :] = (
                b_ref[pl.ds(s * hin, hin), pl.ds(dh * nco, nco)])

        big = jnp.dot(ahcat_ref[...], z_ref[...],
                      preferred_element_type=jnp.float32)
        for co in range(cout):
            o_ref[s, co] = (big[:, co * wout:(co + 1) * wout]
                            + shift_ref[co]).astype(o_ref.dtype)


def _upsample_conv_bn(x, size, conv_w, conv_b, bn_gamma, bn_beta,
                      bn_mean, bn_var, eps=1e-5):
    N, Cin, Hin, Win = x.shape
    Hout, Wout = size
    Cout, Cin_w, KH, KW = conv_w.shape
    assert Cin_w == Cin
    PH, PW = KH // 2, KW // 2
    Wout_p = ((Wout + LANES - 1) // LANES) * LANES

    scale = bn_gamma / jnp.sqrt(bn_var + eps)
    w_eff = conv_w.astype(jnp.float32) * scale[:, None, None, None]
    shift = ((conv_b - bn_mean) * scale + bn_beta).astype(jnp.float32)

    aw = _interp_matrix(Win, Wout)
    aw_pad = jnp.pad(aw, ((PW, PW + (Wout_p - Wout)), (0, 0)))
    aw5 = jnp.stack([aw_pad[dw:dw + Wout_p, :] for dw in range(KW)])
    rwbig = jnp.einsum('abcd,def->bfcae', w_eff, aw5)
    rwbig = rwbig.reshape(Cin * Win, KH * Cout * Wout_p)

    ah_pad = jnp.pad(_interp_matrix(Hin, Hout), ((PH, PH), (0, 0)))
    ahcat = jnp.concatenate([ah_pad[dh:dh + Hout, :] for dh in range(KH)],
                            axis=1)

    xrow = x.transpose(0, 2, 1, 3).reshape(N, Hin, Cin * Win)

    BS = 4 if N % 4 == 0 else 1

    kernel_fn = functools.partial(
        _fused_kernel, bs=BS, cin=Cin, cout=Cout, hin=Hin, hout=Hout,
        wout=Wout_p, kh=KH, kw=KW)

    flops = N * (2 * Hin * (Cin * Win) * KH * Cout * Wout_p
                 + 2 * Hout * KH * Hin * Cout * Wout_p)
    bytes_accessed = 4 * (xrow.size + rwbig.size + ahcat.size
                          + shift.size + N * Cout * Hout * Wout_p)

    out = pl.pallas_call(
        kernel_fn,
        out_shape=jax.ShapeDtypeStruct((N, Cout, Hout, Wout_p), x.dtype),
        grid_spec=pltpu.PrefetchScalarGridSpec(
            num_scalar_prefetch=0,
            grid=(N // BS,),
            in_specs=[
                pl.BlockSpec((BS, Hin, Cin * Win), lambda n: (n, 0, 0)),
                pl.BlockSpec((Cin * Win, KH * Cout * Wout_p),
                             lambda n: (0, 0)),
                pl.BlockSpec((Hout, KH * Hin), lambda n: (0, 0)),
                pl.BlockSpec(memory_space=pltpu.MemorySpace.SMEM),
            ],
            out_specs=pl.BlockSpec((BS, Cout, Hout, Wout_p),
                                   lambda n: (n, 0, 0, 0)),
            scratch_shapes=[
                pltpu.VMEM((BS * Hin, KH * Cout * Wout_p), jnp.float32),
                pltpu.VMEM((KH * Hin, Cout * Wout_p), jnp.float32),
            ],
        ),
        compiler_params=pltpu.CompilerParams(
            dimension_semantics=("parallel",),
            vmem_limit_bytes=64 * 1024 * 1024),
        cost_estimate=pl.CostEstimate(
            flops=flops, transcendentals=0, bytes_accessed=bytes_accessed),
    )(xrow, rwbig, ahcat, shift)

    return out[..., :Wout]


def kernel(x, conv_w, conv_b, bn_gamma, bn_beta, bn_mean, bn_var):
    return _upsample_conv_bn(x, (128, 128), conv_w, conv_b,
                             bn_gamma, bn_beta, bn_mean, bn_var)


# speedup vs baseline: 48.6384x; 1.1904x over previous
import functools

import jax
import jax.numpy as jnp
from jax.experimental import pallas as pl
from jax.experimental.pallas import tpu as pltpu

LANES = 128


def _src_coords(n_in, n_out):
    if n_out == 1:
        return jnp.zeros((1,), jnp.float32)
    return jnp.arange(n_out, dtype=jnp.float32) * ((n_in - 1) / (n_out - 1))


def _interp_matrix(n_in, n_out):
    src = _src_coords(n_in, n_out)
    i0 = jnp.clip(jnp.floor(src).astype(jnp.int32), 0, n_in - 1)
    i1 = jnp.minimum(i0 + 1, n_in - 1)
    frac = src - i0.astype(jnp.float32)
    rows = jnp.arange(n_out)
    a = jnp.zeros((n_out, n_in), jnp.float32)
    a = a.at[rows, i0].add(1.0 - frac)
    a = a.at[rows, i1].add(frac)
    return a


def _fused_kernel(x_ref, rw_ref, ahcat_ref, shift_ref, o_ref,
                  b_ref, z_ref, *, bs, cin, cout, hin, hout, wout, kh, kw):

    nco = cout * wout

    b_ref[...] = jnp.dot(x_ref[...].reshape(bs * hin, -1), rw_ref[...],
                         preferred_element_type=jnp.float32)

    for s in range(bs):
        for dh in range(kh):
            z_ref[pl.ds(dh * hin, hin), :] = (
                b_ref[pl.ds(s * hin, hin),
                      pl.ds(dh * nco, nco)].astype(z_ref.dtype))

        big = jnp.dot(ahcat_ref[...], z_ref[...],
                      preferred_element_type=jnp.float32)
        for co in range(cout):
            o_ref[s, co] = (big[:, co * wout:(co + 1) * wout]
                            + shift_ref[co]).astype(o_ref.dtype)


def _upsample_conv_bn(x, size, conv_w, conv_b, bn_gamma, bn_beta,
                      bn_mean, bn_var, eps=1e-5):
    N, Cin, Hin, Win = x.shape
    Hout, Wout = size
    Cout, Cin_w, KH, KW = conv_w.shape
    assert Cin_w == Cin
    PH, PW = KH // 2, KW // 2
    Wout_p = ((Wout + LANES - 1) // LANES) * LANES

    scale = bn_gamma / jnp.sqrt(bn_var + eps)
    w_eff = conv_w.astype(jnp.float32) * scale[:, None, None, None]
    shift = ((conv_b - bn_mean) * scale + bn_beta).astype(jnp.float32)

    aw = _interp_matrix(Win, Wout)
    aw_pad = jnp.pad(aw, ((PW, PW + (Wout_p - Wout)), (0, 0)))
    aw5 = jnp.stack([aw_pad[dw:dw + Wout_p, :] for dw in range(KW)])
    rwbig = jnp.einsum('abcd,def->bfcae', w_eff, aw5)
    rwbig = rwbig.reshape(Cin * Win, KH * Cout * Wout_p)

    ah_pad = jnp.pad(_interp_matrix(Hin, Hout), ((PH, PH), (0, 0)))
    ahcat = jnp.concatenate([ah_pad[dh:dh + Hout, :] for dh in range(KH)],
                            axis=1)

    xrow = x.transpose(0, 2, 1, 3).reshape(N, Hin, Cin * Win)
    xrow = xrow.astype(jnp.bfloat16)
    rwbig = rwbig.astype(jnp.bfloat16)
    ahcat = ahcat.astype(jnp.bfloat16)

    BS = 8 if N % 8 == 0 else 1

    kernel_fn = functools.partial(
        _fused_kernel, bs=BS, cin=Cin, cout=Cout, hin=Hin, hout=Hout,
        wout=Wout_p, kh=KH, kw=KW)

    flops = N * (2 * Hin * (Cin * Win) * KH * Cout * Wout_p
                 + 2 * Hout * KH * Hin * Cout * Wout_p)
    bytes_accessed = 4 * (xrow.size + rwbig.size + ahcat.size
                          + shift.size + N * Cout * Hout * Wout_p)

    out = pl.pallas_call(
        kernel_fn,
        out_shape=jax.ShapeDtypeStruct((N, Cout, Hout, Wout_p), x.dtype),
        grid_spec=pltpu.PrefetchScalarGridSpec(
            num_scalar_prefetch=0,
            grid=(N // BS,),
            in_specs=[
                pl.BlockSpec((BS, Hin, Cin * Win), lambda n: (n, 0, 0)),
                pl.BlockSpec((Cin * Win, KH * Cout * Wout_p),
                             lambda n: (0, 0)),
                pl.BlockSpec((Hout, KH * Hin), lambda n: (0, 0)),
                pl.BlockSpec(memory_space=pltpu.MemorySpace.SMEM),
            ],
            out_specs=pl.BlockSpec((BS, Cout, Hout, Wout_p),
                                   lambda n: (n, 0, 0, 0)),
            scratch_shapes=[
                pltpu.VMEM((BS * Hin, KH * Cout * Wout_p), jnp.float32),
                pltpu.VMEM((KH * Hin, Cout * Wout_p), jnp.bfloat16),
            ],
        ),
        compiler_params=pltpu.CompilerParams(
            dimension_semantics=("parallel",),
            vmem_limit_bytes=64 * 1024 * 1024),
        cost_estimate=pl.CostEstimate(
            flops=flops, transcendentals=0, bytes_accessed=bytes_accessed),
    )(xrow, rwbig, ahcat, shift)

    return out[..., :Wout]


def kernel(x, conv_w, conv_b, bn_gamma, bn_beta, bn_mean, bn_var):
    return _upsample_conv_bn(x, (128, 128), conv_w, conv_b,
                             bn_gamma, bn_beta, bn_mean, bn_var)
